# baseline (speedup 1.0000x reference)
import functools

import jax
import jax.numpy as jnp
from jax import lax
from jax.experimental import pallas as pl
from jax.experimental.pallas import tpu as pltpu
from jax.experimental.pallas import tpu_sc as plsc

N = 10000
C = 128
E = 320000
B = 2
T = 4
NC = 2
NS = 16
L = 16
EPT = E // (NC * NS)
SW = 128
SROW = 79
EPP = SROW * SW
EWG = 32
NWING = EPP // EWG
WPC = 4
RBUF = 2
NSLOT = 8
WBUF = 5
WEW = 80
WNW = 125
DMASK = 16383
NP = 10240
RPT = NP // NS
BN = 2048
G = NP // BN

_mesh = plsc.VectorSubcoreMesh(core_axis_name="c", subcore_axis_name="s")


@functools.partial(
    pl.kernel,
    mesh=_mesh,
    out_type=jax.ShapeDtypeStruct((NC * NS, WNW, WEW), jnp.float32),
    scratch_types=[
        pltpu.VMEM((WNW, WEW), jnp.int32),
        pltpu.VMEM((WNW, WEW), jnp.int32),
        pltpu.VMEM((WNW, WEW), jnp.float32),
    ] + [pltpu.SemaphoreType.DMA] * WBUF,
)
def _we_gather(adj_hbm, src_hbm, dst_hbm, out_hbm, iv, dv, wb, *sems):
    cid = lax.axis_index("c")
    sid = lax.axis_index("s")
    wid = cid * NS + sid

    pltpu.sync_copy(src_hbm.at[wid], iv)
    pltpu.sync_copy(dst_hbm.at[wid], dv)

    def flat(r, carry):
        for k in range(WEW // L):
            sl = pl.ds(k * L, L)
            iv[r, sl] = iv[r, sl] * N + dv[r, sl]
        return carry

    lax.fori_loop(0, WNW, flat, 0)

    for b in range(WBUF):
        pltpu.async_copy(adj_hbm.at[iv.at[b]], wb.at[b], sems[b])

    def chunk(w0):
        for b in range(WBUF):
            w = w0 + b
            pltpu.make_async_copy(adj_hbm.at[iv.at[w]], wb.at[w], sems[b]).wait()

            @pl.when(w + WBUF < WNW)
            def _():
                pltpu.async_copy(adj_hbm.at[iv.at[w + WBUF]], wb.at[w + WBUF], sems[b])

    pl.loop(0, WNW, step=WBUF)(chunk)
    pltpu.sync_copy(wb, out_hbm.at[wid])


@functools.partial(
    pl.kernel,
    mesh=_mesh,
    out_type=jax.ShapeDtypeStruct((NC, NP, C), jnp.float32),
    scratch_types=[
        pltpu.VMEM((SROW, SW), jnp.int32),
        pltpu.VMEM((SROW, SW), jnp.float32),
        pltpu.VMEM((NSLOT, EWG), jnp.int32),
        pltpu.VMEM((NSLOT, EWG), jnp.int32),
        pltpu.VMEM((RBUF, EWG, C), jnp.float32),
        pltpu.VMEM((RBUF, EWG, C), jnp.float32),
        pltpu.VMEM_SHARED((NP, C), jnp.float32),
    ] + [pltpu.SemaphoreType.DMA] * (2 * RBUF),
)
def _spmm(sd_hbm, we_hbm, h_hbm, zer_hbm, out_hbm,
          sd2, we2, srcr, dstr, gbuf, sbuf, acc, *sems):
    semg = sems[:RBUF]
    sems2 = sems[RBUF:]
    cid = lax.axis_index("c")
    sid = lax.axis_index("s")
    wid = cid * NS + sid
    r0 = sid * RPT

    pltpu.sync_copy(zer_hbm, acc.at[pl.ds(r0, RPT)])
    pltpu.sync_copy(sd_hbm.at[wid], sd2)
    pltpu.sync_copy(we_hbm.at[wid], we2)
    plsc.subcore_barrier()

    def unpack_idx(row, colbase, slot):
        def ubody(g, c2):
            v = sd2[row, pl.ds(colbase + g * L, L)]
            sl = pl.ds(g * L, L)
            srcr[slot, sl] = lax.shift_right_logical(v, 14)
            dstr[slot, sl] = jnp.bitwise_and(v, DMASK)
            return c2

        lax.fori_loop(0, EWG // L, ubody, 0)

    for b in range(RBUF):
        unpack_idx(0, EWG * b, b)
        pltpu.async_copy(h_hbm.at[srcr.at[b]], gbuf.at[b], semg[b])

    def chunk(w0):
        r = w0 // WPC
        for b in range(WPC):
            w = w0 + b
            gb = b % RBUF
            sw = jnp.bitwise_and(w, NSLOT - 1)
            pltpu.make_async_copy(
                h_hbm.at[srcr.at[sw]], gbuf.at[gb], semg[gb]).wait()

            if b >= RBUF:
                pltpu.make_async_copy(
                    sbuf.at[gb], acc.at[dstr.at[sw]], sems2[gb]).wait()
            else:
                @pl.when(w0 > 0)
                def _(gb=gb, sw=sw):
                    pltpu.make_async_copy(
                        sbuf.at[gb], acc.at[dstr.at[sw]], sems2[gb]).wait()

            def sbody(g, c2, _gb=gb, _b=b, _r=r):
                w16 = we2[_r, pl.ds(EWG * _b + g * L, L)]
                for k in range(L):
                    wvec = jnp.broadcast_to(w16[k], (L,))
                    e = g * L + k
                    for c in range(C // L):
                        sl = pl.ds(c * L, L)
                        sbuf[_gb, e, sl] = gbuf[_gb, e, sl] * wvec
                return c2

            lax.fori_loop(0, EWG // L, sbody, 0)
            pltpu.async_copy(sbuf.at[gb], acc.at[dstr.at[sw]], sems2[gb], add=True)

            @pl.when(w + RBUF < NWING)
            def _(b=b, r=r, gb=gb):
                s2 = jnp.bitwise_and(w0 + b + RBUF, NSLOT - 1)
                if b + RBUF < WPC:
                    unpack_idx(r, EWG * (b + RBUF), s2)
                else:
                    unpack_idx(r + 1, EWG * (b + RBUF - WPC), s2)
                pltpu.async_copy(h_hbm.at[srcr.at[s2]], gbuf.at[gb], semg[gb])

    pl.loop(0, NWING, step=WPC)(chunk)
    for b in range(RBUF):
        pltpu.make_async_copy(sbuf.at[b], acc.at[dstr.at[b]], sems2[b]).wait()
    plsc.subcore_barrier()
    pltpu.sync_copy(acc.at[pl.ds(r0, RPT)], out_hbm.at[cid, pl.ds(r0, RPT)])


def _mm_relu_body(p_ref, w_ref, b_ref, o_ref):
    s = p_ref[0] + p_ref[1]
    o_ref[...] = jnp.maximum(
        jnp.dot(s, w_ref[...], preferred_element_type=jnp.float32) + b_ref[...], 0.0)


def _mm_relu(p, W, b2d):
    return pl.pallas_call(
        _mm_relu_body,
        grid=(G,),
        in_specs=[
            pl.BlockSpec((NC, BN, C), lambda i: (0, i, 0)),
            pl.BlockSpec((C, C), lambda i: (0, 0)),
            pl.BlockSpec((1, C), lambda i: (0, 0)),
        ],
        out_specs=pl.BlockSpec((BN, C), lambda i: (i, 0)),
        out_shape=jax.ShapeDtypeStruct((NP, C), jnp.float32),
    )(p, W, b2d)


def _mm_relu_rsum_body(p_ref, w_ref, b_ref, o_ref):
    i = pl.program_id(0)
    s = p_ref[0] + p_ref[1]
    h = jnp.maximum(
        jnp.dot(s, w_ref[...], preferred_element_type=jnp.float32) + b_ref[...], 0.0)
    row = jax.lax.broadcasted_iota(jnp.int32, (BN, 1), 0) + i * BN
    h = jnp.where(row < N, h, 0.0)
    o_ref[pl.ds(i, 1), :] = jnp.sum(h, axis=0, keepdims=True)


def _mm_relu_rsum(p, W, b2d):
    return pl.pallas_call(
        _mm_relu_rsum_body,
        grid=(G,),
        in_specs=[
            pl.BlockSpec((NC, BN, C), lambda i: (0, i, 0)),
            pl.BlockSpec((C, C), lambda i: (0, 0)),
            pl.BlockSpec((1, C), lambda i: (0, 0)),
        ],
        out_specs=pl.BlockSpec((G, C), lambda i: (0, 0)),
        out_shape=jax.ShapeDtypeStruct((G, C), jnp.float32),
    )(p, W, b2d)


def _heads_body(sp_ref, wl1, bl1, wl2, bl2, wd1, bd1, wd2, bd2, cls_ref, dom_ref):
    seq = jnp.sum(sp_ref[...], axis=2) * (1.0 / N)
    m = jnp.mean(seq, axis=1)
    var = jnp.sum((seq - m[:, None, :]) ** 2, axis=1) * (1.0 / (T - 1))
    sd = jnp.sqrt(var)
    feat = jnp.concatenate([m, sd], axis=-1)
    hl = jnp.maximum(
        jnp.dot(feat, wl1[...], preferred_element_type=jnp.float32) + bl1[...], 0.0)
    cls_ref[...] = jnp.dot(hl, wl2[...], preferred_element_type=jnp.float32) + bl2[...]
    hd = jnp.maximum(
        jnp.dot(feat, wd1[...], preferred_element_type=jnp.float32) + bd1[...], 0.0)
    dom_ref[...] = jnp.dot(hd, wd2[...], preferred_element_type=jnp.float32) + bd2[...]


def _heads(sp, Wl1, bl1, Wl2, bl2, Wd1, bd1, Wd2, bd2):
    return pl.pallas_call(
        _heads_body,
        out_shape=(
            jax.ShapeDtypeStruct((B, 2), jnp.float32),
            jax.ShapeDtypeStruct((B, 2), jnp.float32),
        ),
    )(sp, Wl1, bl1.reshape(1, C), Wl2, bl2.reshape(1, 2),
      Wd1, bd1.reshape(1, C), Wd2, bd2.reshape(1, 2))


def kernel(x, base_adj, edge_index, W1, b1, W2, b2,
           Wl1, bl1, Wl2, bl2, Wd1, bd1, Wd2, bd2):
    src2d = edge_index[0].reshape(NC * NS, EPT)
    dst2d = edge_index[1].reshape(NC * NS, EPT)
    adj_flat = base_adj.reshape(N * N)
    w_e = _we_gather(adj_flat, src2d.reshape(NC * NS, WNW, WEW),
                     dst2d.reshape(NC * NS, WNW, WEW))

    packed = jnp.bitwise_or(jnp.left_shift(src2d, 14), dst2d)
    sd3 = jnp.pad(packed, ((0, 0), (0, EPP - EPT))).reshape(NC * NS, SROW, SW)
    we3 = jnp.pad(w_e.reshape(NC * NS, EPT),
                  ((0, 0), (0, EPP - EPT))).reshape(NC * NS, SROW, SW)
    zer = jnp.zeros((RPT, C), jnp.float32)

    x8 = jnp.pad(x.reshape(B * T, N, C), ((0, 0), (0, NP - N), (0, 0)))
    b1_2d = b1.reshape(1, C)
    b2_2d = b2.reshape(1, C)

    h1 = []
    for s in range(B * T):
        p = _spmm(sd3, we3, x8[s], zer)
        h1.append(_mm_relu(p, W1, b1_2d))

    parts = []
    for s in range(B * T):
        p = _spmm(sd3, we3, h1[s], zer)
        parts.append(_mm_relu_rsum(p, W2, b2_2d))

    sp = jnp.stack(parts).reshape(B, T, G, C)
    return _heads(sp, Wl1, bl1, Wl2, bl2, Wd1, bd1, Wd2, bd2)


# speedup vs baseline: 1.0692x; 1.0692x over previous
import functools

import jax
import jax.numpy as jnp
from jax import lax
from jax.experimental import pallas as pl
from jax.experimental.pallas import tpu as pltpu
from jax.experimental.pallas import tpu_sc as plsc

N = 10000
C = 128
E = 320000
B = 2
T = 4
NC = 2
NS = 16
L = 16
EPT = E // (NC * NS)
SW = 128
SROW = 79
EPP = SROW * SW
EWG = 64
NWING = EPP // EWG
RBUF = 2
NSLOT = 4
WBUF = 5
WEW = 80
WNW = 125
DMASK = 16383
NP = 10240
RPT = NP // NS
BN = 2048
G = NP // BN

_mesh = plsc.VectorSubcoreMesh(core_axis_name="c", subcore_axis_name="s")


@functools.partial(
    pl.kernel,
    mesh=_mesh,
    out_type=jax.ShapeDtypeStruct((NC * NS, WNW, WEW), jnp.float32),
    scratch_types=[
        pltpu.VMEM((WNW, WEW), jnp.int32),
        pltpu.VMEM((WNW, WEW), jnp.int32),
        pltpu.VMEM((WNW, WEW), jnp.float32),
    ] + [pltpu.SemaphoreType.DMA] * WBUF,
)
def _we_gather(adj_hbm, src_hbm, dst_hbm, out_hbm, iv, dv, wb, *sems):
    cid = lax.axis_index("c")
    sid = lax.axis_index("s")
    wid = cid * NS + sid

    pltpu.sync_copy(src_hbm.at[wid], iv)
    pltpu.sync_copy(dst_hbm.at[wid], dv)

    def flat(r, carry):
        for k in range(WEW // L):
            sl = pl.ds(k * L, L)
            iv[r, sl] = iv[r, sl] * N + dv[r, sl]
        return carry

    lax.fori_loop(0, WNW, flat, 0)

    for b in range(WBUF):
        pltpu.async_copy(adj_hbm.at[iv.at[b]], wb.at[b], sems[b])

    def chunk(w0):
        for b in range(WBUF):
            w = w0 + b
            pltpu.make_async_copy(adj_hbm.at[iv.at[w]], wb.at[w], sems[b]).wait()

            @pl.when(w + WBUF < WNW)
            def _():
                pltpu.async_copy(adj_hbm.at[iv.at[w + WBUF]], wb.at[w + WBUF], sems[b])

    pl.loop(0, WNW, step=WBUF)(chunk)
    pltpu.sync_copy(wb, out_hbm.at[wid])


@functools.partial(
    pl.kernel,
    mesh=_mesh,
    out_type=jax.ShapeDtypeStruct((B * T, NC, NP, C), jnp.float32),
    scratch_types=[
        pltpu.VMEM((SROW, SW), jnp.int32),
        pltpu.VMEM((SROW, SW), jnp.float32),
        pltpu.VMEM((NSLOT, EWG), jnp.int32),
        pltpu.VMEM((NSLOT, EWG), jnp.int32),
        pltpu.VMEM((RBUF, EWG, C), jnp.float32),
        pltpu.VMEM_SHARED((NP, C), jnp.float32),
    ] + [pltpu.SemaphoreType.DMA] * RBUF,
)
def _spmm8(sd_hbm, we_hbm, h_hbm, zer_hbm, out_hbm,
           sd2, we2, srcr, dstr, rows, acc, *sems):
    cid = lax.axis_index("c")
    sid = lax.axis_index("s")
    wid = cid * NS + sid
    r0 = sid * RPT

    pltpu.sync_copy(sd_hbm.at[wid], sd2)
    pltpu.sync_copy(we_hbm.at[wid], we2)

    def step(s, carry):
        soff = s * NP

        def unpack_idx(row, half, slot):
            def ubody(g, c2):
                v = sd2[row, pl.ds(64 * half + g * L, L)]
                sl = pl.ds(g * L, L)
                srcr[slot, sl] = lax.shift_right_logical(v, 14) + soff
                dstr[slot, sl] = jnp.bitwise_and(v, DMASK)
                return c2

            lax.fori_loop(0, EWG // L, ubody, 0)

        pltpu.sync_copy(zer_hbm, acc.at[pl.ds(r0, RPT)])
        plsc.subcore_barrier()

        for b in range(RBUF):
            unpack_idx(0, b, b)
            pltpu.async_copy(h_hbm.at[srcr.at[b]], rows.at[b], sems[b])

        def chunk(w0):
            r = w0 // 2
            for b in range(RBUF):
                w = w0 + b
                sw = jnp.bitwise_and(w, NSLOT - 1)
                pltpu.make_async_copy(
                    h_hbm.at[srcr.at[sw]], rows.at[b], sems[b]).wait()

                def sbody(g, c2, _b=b, _r=r):
                    w16 = we2[_r, pl.ds(64 * _b + g * L, L)]
                    for k in range(L):
                        wvec = jnp.broadcast_to(w16[k], (L,))
                        e = g * L + k
                        for c in range(C // L):
                            sl = pl.ds(c * L, L)
                            rows[_b, e, sl] = rows[_b, e, sl] * wvec
                    return c2

                lax.fori_loop(0, EWG // L, sbody, 0)
                pltpu.sync_copy(rows.at[b], acc.at[dstr.at[sw]], add=True)

                @pl.when(w + RBUF < NWING)
                def _(b=b, w=w, r=r):
                    s2 = jnp.bitwise_and(w + RBUF, NSLOT - 1)
                    unpack_idx(r + 1, b, s2)
                    pltpu.async_copy(h_hbm.at[srcr.at[s2]], rows.at[b], sems[b])

        pl.loop(0, NWING, step=RBUF)(chunk)
        plsc.subcore_barrier()
        pltpu.sync_copy(acc.at[pl.ds(r0, RPT)],
                        out_hbm.at[s, cid, pl.ds(r0, RPT)])
        return carry

    lax.fori_loop(0, B * T, step, 0)


def _mm_relu_body(p_ref, w_ref, b_ref, o_ref):
    s = p_ref[0, 0] + p_ref[0, 1]
    o_ref[0] = jnp.maximum(
        jnp.dot(s, w_ref[...], preferred_element_type=jnp.float32) + b_ref[...], 0.0)


def _mm_relu(p, W, b2d):
    return pl.pallas_call(
        _mm_relu_body,
        grid=(B * T, G),
        in_specs=[
            pl.BlockSpec((1, NC, BN, C), lambda s, i: (s, 0, i, 0)),
            pl.BlockSpec((C, C), lambda s, i: (0, 0)),
            pl.BlockSpec((1, C), lambda s, i: (0, 0)),
        ],
        out_specs=pl.BlockSpec((1, BN, C), lambda s, i: (s, i, 0)),
        out_shape=jax.ShapeDtypeStruct((B * T, NP, C), jnp.float32),
    )(p, W, b2d)


def _mm_relu_rsum_body(p_ref, w_ref, b_ref, o_ref):
    i = pl.program_id(1)
    s = p_ref[0, 0] + p_ref[0, 1]
    h = jnp.maximum(
        jnp.dot(s, w_ref[...], preferred_element_type=jnp.float32) + b_ref[...], 0.0)
    row = jax.lax.broadcasted_iota(jnp.int32, (BN, 1), 0) + i * BN
    h = jnp.where(row < N, h, 0.0)
    o_ref[0, pl.ds(i, 1), :] = jnp.sum(h, axis=0, keepdims=True)


def _mm_relu_rsum(p, W, b2d):
    return pl.pallas_call(
        _mm_relu_rsum_body,
        grid=(B * T, G),
        in_specs=[
            pl.BlockSpec((1, NC, BN, C), lambda s, i: (s, 0, i, 0)),
            pl.BlockSpec((C, C), lambda s, i: (0, 0)),
            pl.BlockSpec((1, C), lambda s, i: (0, 0)),
        ],
        out_specs=pl.BlockSpec((1, G, C), lambda s, i: (s, 0, 0)),
        out_shape=jax.ShapeDtypeStruct((B * T, G, C), jnp.float32),
    )(p, W, b2d)


def _heads_body(sp_ref, wl1, bl1, wl2, bl2, wd1, bd1, wd2, bd2, cls_ref, dom_ref):
    seq = jnp.sum(sp_ref[...], axis=2) * (1.0 / N)
    m = jnp.mean(seq, axis=1)
    var = jnp.sum((seq - m[:, None, :]) ** 2, axis=1) * (1.0 / (T - 1))
    sd = jnp.sqrt(var)
    feat = jnp.concatenate([m, sd], axis=-1)
    hl = jnp.maximum(
        jnp.dot(feat, wl1[...], preferred_element_type=jnp.float32) + bl1[...], 0.0)
    cls_ref[...] = jnp.dot(hl, wl2[...], preferred_element_type=jnp.float32) + bl2[...]
    hd = jnp.maximum(
        jnp.dot(feat, wd1[...], preferred_element_type=jnp.float32) + bd1[...], 0.0)
    dom_ref[...] = jnp.dot(hd, wd2[...], preferred_element_type=jnp.float32) + bd2[...]


def _heads(sp, Wl1, bl1, Wl2, bl2, Wd1, bd1, Wd2, bd2):
    return pl.pallas_call(
        _heads_body,
        out_shape=(
            jax.ShapeDtypeStruct((B, 2), jnp.float32),
            jax.ShapeDtypeStruct((B, 2), jnp.float32),
        ),
    )(sp, Wl1, bl1.reshape(1, C), Wl2, bl2.reshape(1, 2),
      Wd1, bd1.reshape(1, C), Wd2, bd2.reshape(1, 2))


def kernel(x, base_adj, edge_index, W1, b1, W2, b2,
           Wl1, bl1, Wl2, bl2, Wd1, bd1, Wd2, bd2):
    src2d = edge_index[0].reshape(NC * NS, EPT)
    dst2d = edge_index[1].reshape(NC * NS, EPT)
    adj_flat = base_adj.reshape(N * N)
    w_e = _we_gather(adj_flat, src2d.reshape(NC * NS, WNW, WEW),
                     dst2d.reshape(NC * NS, WNW, WEW))

    packed = jnp.bitwise_or(jnp.left_shift(src2d, 14), dst2d)
    sd3 = jnp.pad(packed, ((0, 0), (0, EPP - EPT))).reshape(NC * NS, SROW, SW)
    we3 = jnp.pad(w_e.reshape(NC * NS, EPT),
                  ((0, 0), (0, EPP - EPT))).reshape(NC * NS, SROW, SW)
    zer = jnp.zeros((RPT, C), jnp.float32)

    x8 = jnp.pad(x.reshape(B * T, N, C), ((0, 0), (0, NP - N), (0, 0)))
    b1_2d = b1.reshape(1, C)
    b2_2d = b2.reshape(1, C)

    p1 = _spmm8(sd3, we3, x8.reshape(B * T * NP, C), zer)
    h1 = _mm_relu(p1, W1, b1_2d)
    p2 = _spmm8(sd3, we3, h1.reshape(B * T * NP, C), zer)
    parts = _mm_relu_rsum(p2, W2, b2_2d)

    sp = parts.reshape(B, T, G, C)
    return _heads(sp, Wl1, bl1, Wl2, bl2, Wd1, bd1, Wd2, bd2)
